# baseline (speedup 1.0000x reference)
import functools

import jax
import jax.numpy as jnp
from jax import lax
from jax.experimental import pallas as pl
from jax.experimental.pallas import tpu as pltpu
from jax.experimental.pallas import tpu_sc as plsc

_B = 128
_N = 32768
_K = 64
_L = 16
_NBINS = 2048
_LOWBITS = 21
_LOWMASK = (1 << _LOWBITS) - 1
_NW = 32
_ROWS_PER_W = _B // _NW
_NV = _N // _L
_UNROLL = 8


def _keys(iv):
    kv = iv ^ ((iv >> 31) & jnp.int32(0x7FFFFFFF))
    return jnp.where(iv == jnp.int32(-2147483647 - 1), jnp.int32(0), kv)


def _sc_body(x_hbm, out_hbm, row_v, hist_v, cand_v):
    wid = lax.axis_index("s") * 2 + lax.axis_index("c")
    lane = lax.broadcasted_iota(jnp.int32, (_L,), 0)
    ones = jnp.ones((_L,), jnp.int32)

    def do_row(r, _):
        row = wid * _ROWS_PER_W + r
        pltpu.sync_copy(x_hbm.at[row], row_v)

        def zero_body(i, _):
            for u in range(_UNROLL):
                hist_v[pl.ds((i * _UNROLL + u) * _L, _L)] = jnp.zeros(
                    (_L,), jnp.int32)
            return 0

        lax.fori_loop(0, (_NBINS * _L) // (_L * _UNROLL), zero_body, 0)

        def hist_body(i, kmax):
            for u in range(_UNROLL):
                iv = row_v[pl.ds((i * _UNROLL + u) * _L, _L)]
                kv = _keys(iv)
                kmax = jnp.maximum(kmax, kv)
                digit = (kv >> 21) + jnp.int32(1024)
                addr = (digit << 4) | lane
                plsc.addupdate_scatter(hist_v, [addr], ones)
            return kmax

        kmax = lax.fori_loop(0, _NV // _UNROLL, hist_body,
                             jnp.full((_L,), -2147483647 - 1, jnp.int32))
        max_key = lax.reduce_max(kmax, (0,))
        bmax = (max_key >> 21) + jnp.int32(1024)

        def bin_sum(b):
            return lax.reduce_sum(hist_v[pl.ds(b * _L, _L)], (0,))

        def walk_cond(c):
            _, above, cnt = c
            return above + cnt < _K

        def walk_body(c):
            b, above, cnt = c
            b2 = b - 1
            return (b2, above + cnt, bin_sum(b2))

        b_t, above, _ = lax.while_loop(
            walk_cond, walk_body, (bmax, jnp.int32(0), bin_sum(bmax)))
        kprime = _K - above

        def compact_body(i, off):
            for u in range(_UNROLL):
                iv = row_v[pl.ds((i * _UNROLL + u) * _L, _L)]
                kv = _keys(iv)
                digit = (kv >> 21) + jnp.int32(1024)
                m = digit == b_t
                pos = plsc.cumsum(jnp.where(m, 1, 0).astype(jnp.int32))
                plsc.store_scatter(cand_v, [off + pos - 1],
                                   kv & jnp.int32(_LOWMASK), mask=m)
                off = off + lax.reduce_max(pos, (0,))
            return off

        n_cand = lax.fori_loop(0, _NV // _UNROLL, compact_body, jnp.int32(0))

        nv_cand = (n_cand + (_L - 1)) // _L

        def count_ge(cand):
            def cbody(i, cv):
                lv = cand_v[pl.ds(i * _L, _L)]
                valid = (i * _L + lane) < n_cand
                m = (lv >= cand) & valid
                return cv + m.astype(jnp.int32)

            cvec = lax.fori_loop(0, nv_cand, cbody, jnp.zeros((_L,), jnp.int32))
            return lax.reduce_sum(cvec, (0,))

        t_low = jnp.int32(0)
        for bit in range(_LOWBITS - 1, -1, -1):
            cand = t_low | jnp.int32(1 << bit)
            t_low = jnp.where(count_ge(cand) >= kprime, cand, t_low)

        thr_key = ((b_t - jnp.int32(1024)) << 21) | t_low

        def mask_body(i, _):
            for u in range(_UNROLL):
                sl = pl.ds((i * _UNROLL + u) * _L, _L)
                kv = _keys(row_v[sl])
                row_v[sl] = jnp.where(kv >= thr_key,
                                      jnp.int32(0x3F800000), jnp.int32(0))
            return 0

        lax.fori_loop(0, _NV // _UNROLL, mask_body, 0)
        pltpu.sync_copy(row_v, out_hbm.at[row])
        return 0

    lax.fori_loop(0, _ROWS_PER_W, do_row, 0)


def kernel(logits):
    x = lax.bitcast_convert_type(jnp.squeeze(logits, axis=1), jnp.int32)
    mesh = plsc.VectorSubcoreMesh(core_axis_name="c", subcore_axis_name="s")
    f = functools.partial(
        pl.kernel,
        mesh=mesh,
        compiler_params=pltpu.CompilerParams(needs_layout_passes=False),
        out_type=jax.ShapeDtypeStruct((_B, _N), jnp.int32),
        scratch_types=[
            pltpu.VMEM((_N,), jnp.int32),
            pltpu.VMEM((_NBINS * _L,), jnp.int32),
            pltpu.VMEM((_N + _L,), jnp.int32),
        ],
    )(_sc_body)
    return lax.bitcast_convert_type(f(x), jnp.float32)


# speedup vs baseline: 2.3566x; 2.3566x over previous
import functools

import jax
import jax.numpy as jnp
from jax import lax
from jax.experimental import pallas as pl
from jax.experimental.pallas import tpu as pltpu
from jax.experimental.pallas import tpu_sc as plsc

_B = 128
_N = 32768
_K = 64
_L = 16
_BINBITS = 10
_NBINS = 1 << _BINBITS
_LOWBITS = 32 - _BINBITS
_LOWMASK = (1 << _LOWBITS) - 1
_NW = 32
_ROWS_PER_W = _B // _NW
_NV = _N // _L
_UNROLL = 8


def _keys(iv):
    kv = iv ^ ((iv >> 31) & jnp.int32(0x7FFFFFFF))
    return jnp.where(iv == jnp.int32(-2147483647 - 1), jnp.int32(0), kv)


def _sc_body(x_hbm, out_hbm, row_v, hist_v, cand_v):
    wid = lax.axis_index("s") * 2 + lax.axis_index("c")
    lane = lax.broadcasted_iota(jnp.int32, (_L,), 0)
    ones = jnp.ones((_L,), jnp.int32)
    zeros = jnp.zeros((_L,), jnp.int32)

    @plsc.parallel_loop(0, _NBINS, unroll=_UNROLL)
    def _(i):
        hist_v[pl.ds(i * _L, _L)] = zeros

    def do_row(r, _):
        row = wid * _ROWS_PER_W + r
        pltpu.sync_copy(x_hbm.at[row], row_v)

        @plsc.parallel_loop(0, _NV, unroll=_UNROLL,
                            carry=jnp.full((_L,), -2147483647 - 1, jnp.int32))
        def kmax(i, acc):
            iv = row_v[pl.ds(i * _L, _L)]
            kv = _keys(iv)
            digit = (kv >> _LOWBITS) + jnp.int32(_NBINS // 2)
            addr = (digit << 4) | lane
            plsc.addupdate_scatter(hist_v, [addr], ones)
            return jnp.maximum(acc, kv)

        max_key = lax.reduce_max(kmax, (0,))
        bmax = (max_key >> _LOWBITS) + jnp.int32(_NBINS // 2)

        def bin_sum(b):
            return lax.reduce_sum(hist_v[pl.ds(b * _L, _L)], (0,))

        def walk_cond(c):
            _, above, cnt = c
            return above + cnt < _K

        def walk_body(c):
            b, above, cnt = c
            b2 = b - 1
            return (b2, above + cnt, bin_sum(b2))

        b_t, above, _ = lax.while_loop(
            walk_cond, walk_body, (bmax, jnp.int32(0), bin_sum(bmax)))
        kprime = _K - above

        @plsc.parallel_loop(0, _NV, unroll=_UNROLL, carry=zeros)
        def n_vec(i, off):
            iv = row_v[pl.ds(i * _L, _L)]
            kv = _keys(iv)
            digit = (kv >> _LOWBITS) + jnp.int32(_NBINS // 2)
            m = digit == b_t
            plsc.store_scatter(cand_v, [(off << 4) | lane],
                               kv & jnp.int32(_LOWMASK), mask=m)
            @pl.when(i < _NBINS)
            def _():
                hist_v[pl.ds(i * _L, _L)] = zeros
            return off + m.astype(jnp.int32)

        nv_cand = lax.reduce_max(n_vec, (0,))

        def count_ge(cand):
            @plsc.parallel_loop(0, nv_cand, unroll=2, carry=zeros)
            def cvec(j, acc):
                lv = cand_v[pl.ds(j * _L, _L)]
                m = (lv >= cand) & (j < n_vec)
                return acc + m.astype(jnp.int32)

            return lax.reduce_sum(cvec, (0,))

        t_low = jnp.int32(0)
        for bit in range(_LOWBITS - 1, -1, -1):
            cand = t_low | jnp.int32(1 << bit)
            t_low = jnp.where(count_ge(cand) >= kprime, cand, t_low)

        thr_key = ((b_t - jnp.int32(_NBINS // 2)) << _LOWBITS) | t_low

        @plsc.parallel_loop(0, _NV, unroll=_UNROLL)
        def _(i):
            sl = pl.ds(i * _L, _L)
            kv = _keys(row_v[sl])
            row_v[sl] = jnp.where(kv >= thr_key,
                                  jnp.int32(0x3F800000), jnp.int32(0))

        pltpu.sync_copy(row_v, out_hbm.at[row])
        return 0

    lax.fori_loop(0, _ROWS_PER_W, do_row, 0)


def kernel(logits):
    x = lax.bitcast_convert_type(jnp.squeeze(logits, axis=1), jnp.int32)
    mesh = plsc.VectorSubcoreMesh(core_axis_name="c", subcore_axis_name="s")
    f = functools.partial(
        pl.kernel,
        mesh=mesh,
        compiler_params=pltpu.CompilerParams(needs_layout_passes=False),
        out_type=jax.ShapeDtypeStruct((_B, _N), jnp.int32),
        scratch_types=[
            pltpu.VMEM((_N,), jnp.int32),
            pltpu.VMEM((_NBINS * _L,), jnp.int32),
            pltpu.VMEM((_N,), jnp.int32),
        ],
    )(_sc_body)
    return lax.bitcast_convert_type(f(x), jnp.float32)


# speedup vs baseline: 2.5807x; 1.0951x over previous
import functools

import jax
import jax.numpy as jnp
from jax import lax
from jax.experimental import pallas as pl
from jax.experimental.pallas import tpu as pltpu
from jax.experimental.pallas import tpu_sc as plsc

_B = 128
_N = 32768
_K = 64
_L = 16
_BINBITS = 10
_NBINS = 1 << _BINBITS
_LOWBITS = 32 - _BINBITS
_NW = 32
_ROWS_PER_W = _B // _NW
_NV = _N // _L
_UNROLL = 8


def _sc_body(x_hbm, out_hbm, row_a, row_b, hist_v, cand_v,
             sem_ia, sem_ib, sem_oa, sem_ob):
    wid = lax.axis_index("s") * 2 + lax.axis_index("c")
    base = wid * _ROWS_PER_W
    lane = lax.broadcasted_iota(jnp.int32, (_L,), 0)
    ones = jnp.ones((_L,), jnp.int32)
    zeros = jnp.zeros((_L,), jnp.int32)
    laneb = lane + jnp.int32((_NBINS // 2) * _L)

    @plsc.parallel_loop(0, _NBINS, unroll=_UNROLL)
    def _(i):
        hist_v[pl.ds(i * _L, _L)] = zeros

    bufs = [row_a, row_b]
    sin = [sem_ia, sem_ib]
    sout = [sem_oa, sem_ob]
    in_h = [None, None]
    out_h = [None, None]
    in_h[0] = pltpu.async_copy(x_hbm.at[base], row_a, sin[0])

    for r in range(_ROWS_PER_W):
        p = r % 2
        q = 1 - p
        row_v = bufs[p]
        in_h[p].wait()

        @plsc.parallel_loop(0, _NV, unroll=_UNROLL,
                            carry=jnp.full((_L,), -(_NBINS // 2), jnp.int32))
        def dmax(i, acc):
            iv = row_v[pl.ds(i * _L, _L)]
            d = (iv >> _LOWBITS) ^ ((iv >> 31) & jnp.int32(0x1FF))
            plsc.addupdate_scatter(hist_v, [(d << 4) + laneb], ones)
            return jnp.maximum(acc, d)

        bmax = lax.reduce_max(dmax, (0,))

        def bin_sum(b):
            return lax.reduce_sum(
                hist_v[pl.ds(b * _L + (_NBINS // 2) * _L, _L)], (0,))

        def walk_cond(c):
            _, above, cnt = c
            return above + cnt < _K

        def walk_body(c):
            b, above, cnt = c
            return (b - 1, above + cnt, bin_sum(b - 1))

        b_t, above, _ = lax.while_loop(
            walk_cond, walk_body, (bmax, jnp.int32(0), bin_sum(bmax)))
        kprime = _K - above

        if r + 1 < _ROWS_PER_W:
            if out_h[q] is not None:
                out_h[q].wait()
            in_h[q] = pltpu.async_copy(x_hbm.at[base + r + 1], bufs[q], sin[q])

        @plsc.parallel_loop(0, _NV, unroll=_UNROLL, carry=zeros)
        def n_vec(i, off):
            iv = row_v[pl.ds(i * _L, _L)]
            kv = iv ^ ((iv >> 31) & jnp.int32(0x7FFFFFFF))
            m = (kv >> _LOWBITS) == b_t
            plsc.store_scatter(cand_v, [(off << 4) | lane], kv, mask=m)
            return off + m.astype(jnp.int32)

        nv_cand = lax.reduce_max(n_vec, (0,))

        def count_ge(cand):
            @plsc.parallel_loop(0, nv_cand, carry=zeros)
            def cvec(j, acc):
                lv = cand_v[pl.ds(j * _L, _L)]
                m = (lv >= cand) & (j < n_vec)
                return acc + m.astype(jnp.int32)

            return lax.reduce_sum(cvec, (0,))

        t_key = b_t << _LOWBITS
        for bit in range(_LOWBITS - 1, -1, -1):
            cand = t_key | jnp.int32(1 << bit)
            t_key = jnp.where(count_ge(cand) >= kprime, cand, t_key)

        one_f = jnp.int32(0x3F800000)

        @pl.when(t_key >= 0)
        def _():
            @plsc.parallel_loop(0, _NV, unroll=_UNROLL)
            def _(i):
                sl = pl.ds(i * _L, _L)
                iv = row_v[sl]
                row_v[sl] = jnp.where(iv >= t_key, one_f, jnp.int32(0))
                hist_v[pl.ds((i & (_NBINS - 1)) * _L, _L)] = zeros

        @pl.when(t_key < 0)
        def _():
            t_raw = t_key ^ jnp.int32(0x7FFFFFFF)

            @plsc.parallel_loop(0, _NV, unroll=_UNROLL)
            def _(i):
                sl = pl.ds(i * _L, _L)
                iv = row_v[sl]
                row_v[sl] = jnp.where((iv >= 0) | (iv <= t_raw),
                                      one_f, jnp.int32(0))
                hist_v[pl.ds((i & (_NBINS - 1)) * _L, _L)] = zeros

        out_h[p] = pltpu.async_copy(row_v, out_hbm.at[base + r], sout[p])

    for p in (0, 1):
        if out_h[p] is not None:
            out_h[p].wait()


def kernel(logits):
    x = lax.bitcast_convert_type(jnp.squeeze(logits, axis=1), jnp.int32)
    mesh = plsc.VectorSubcoreMesh(core_axis_name="c", subcore_axis_name="s")
    f = functools.partial(
        pl.kernel,
        mesh=mesh,
        compiler_params=pltpu.CompilerParams(needs_layout_passes=False),
        out_type=jax.ShapeDtypeStruct((_B, _N), jnp.int32),
        scratch_types=[
            pltpu.VMEM((_N,), jnp.int32),
            pltpu.VMEM((_N,), jnp.int32),
            pltpu.VMEM((_NBINS * _L,), jnp.int32),
            pltpu.VMEM((_N,), jnp.int32),
            pltpu.SemaphoreType.DMA,
            pltpu.SemaphoreType.DMA,
            pltpu.SemaphoreType.DMA,
            pltpu.SemaphoreType.DMA,
        ],
    )(_sc_body)
    return lax.bitcast_convert_type(f(x), jnp.float32)
